# baseline (speedup 1.0000x reference)
import functools

import jax
import jax.numpy as jnp
from jax import lax
from jax.experimental import pallas as pl
from jax.experimental.pallas import tpu as pltpu
from jax.experimental.pallas import tpu_sc as plsc

N = 10000
E = 320000
D_IN = 128
H = 64

NC = 2
NS = 16
NW = NC * NS
L = 16

CH = 128
EC = E // CH
CPT = 80
CPT_LAST = EC - 31 * CPT
EPT_PAD = CPT * CH

GCH = 512
GPB = GCH // CH

NR = 10240
RPT = NR // NS
IPT = N // NS - 16

BR = N

_mesh = plsc.VectorSubcoreMesh(core_axis_name="c", subcore_axis_name="s")
_sc_params = pltpu.CompilerParams(use_tc_tiling_on_sc=False)


@functools.partial(
    pl.kernel,
    out_type=jax.ShapeDtypeStruct((NC, NR, L), jnp.float32),
    mesh=_mesh,
    compiler_params=_sc_params,
    scratch_types=[
        pltpu.VMEM((CPT, CH), jnp.int32),
        pltpu.VMEM((CH, L), jnp.float32),
        [pltpu.SemaphoreType.DMA for _ in range(4)],
        pltpu.VMEM_SHARED((NR, L), jnp.float32),
    ],
)
def _hist_kernel(dst_hbm, zero_hbm, ones_hbm, out_hbm, dstv, buf, hsems, sdeg):
    c = lax.axis_index("c")
    s = lax.axis_index("s")
    w = s * NC + c
    nch = jnp.where(w == NW - 1, CPT_LAST, CPT)

    @pl.when(w < NW - 1)
    def _():
        pltpu.sync_copy(dst_hbm.at[pl.ds(w * CPT, CPT)], dstv)

    @pl.when(w == NW - 1)
    def _():
        pltpu.sync_copy(dst_hbm.at[pl.ds(w * CPT, CPT_LAST)], dstv.at[pl.ds(0, CPT_LAST)])

    base = s * RPT
    pltpu.sync_copy(zero_hbm, sdeg.at[pl.ds(base, RPT)])
    pltpu.sync_copy(ones_hbm, buf)

    plsc.subcore_barrier()

    @pl.loop(0, nch // 4)
    def _chunkgrp(q):
        j = q * 4
        for r in range(4):
            @pl.when(q > 0)
            def _():
                pltpu.make_async_copy(buf, sdeg.at[dstv.at[0]], hsems[r]).wait()

            pltpu.async_copy(buf, sdeg.at[dstv.at[j + r]], hsems[r], add=True)

    for r in range(4):
        pltpu.make_async_copy(buf, sdeg.at[dstv.at[0]], hsems[r]).wait()

    plsc.subcore_barrier()
    pltpu.sync_copy(sdeg.at[pl.ds(base, RPT)], out_hbm.at[c].at[pl.ds(base, RPT)])


@functools.partial(
    pl.kernel,
    out_type=jax.ShapeDtypeStruct((NC, NR, H), jnp.float32),
    mesh=_mesh,
    compiler_params=_sc_params,
    scratch_types=[
        pltpu.VMEM((CPT, CH), jnp.int32),
        pltpu.VMEM((CPT, CH), jnp.int32),
        [pltpu.VMEM((CH, H), jnp.float32) for _ in range(8)],
        [pltpu.SemaphoreType.DMA for _ in range(8)],
        [pltpu.SemaphoreType.DMA for _ in range(8)],
        pltpu.VMEM_SHARED((NR, H), jnp.float32),
    ],
)
def _scatter_kernel(g_hbm, src_hbm, dst_hbm, out_hbm, srcv, dstv, bufs, gsems, ssems, acc):
    c = lax.axis_index("c")
    s = lax.axis_index("s")
    w = s * NC + c
    nch = jnp.where(w == NW - 1, CPT_LAST, CPT)

    @pl.when(w < NW - 1)
    def _():
        pltpu.sync_copy(src_hbm.at[pl.ds(w * CPT, CPT)], srcv)
        pltpu.sync_copy(dst_hbm.at[pl.ds(w * CPT, CPT)], dstv)

    @pl.when(w == NW - 1)
    def _():
        pltpu.sync_copy(src_hbm.at[pl.ds(w * CPT, CPT_LAST)], srcv.at[pl.ds(0, CPT_LAST)])
        pltpu.sync_copy(dst_hbm.at[pl.ds(w * CPT, CPT_LAST)], dstv.at[pl.ds(0, CPT_LAST)])

    ibase = s * IPT
    pltpu.sync_copy(g_hbm.at[pl.ds(ibase, IPT)], acc.at[pl.ds(ibase, IPT)])

    @pl.when(s == NS - 1)
    def _():
        pltpu.sync_copy(g_hbm.at[pl.ds(NS * IPT, N - NS * IPT)],
                        acc.at[pl.ds(NS * IPT, N - NS * IPT)])

    plsc.subcore_barrier()

    def _ring(nck, R):
        for r in range(R - 1):
            pltpu.async_copy(g_hbm.at[srcv.at[r]], bufs[r], gsems[r])

        @pl.loop(0, nck // R)
        def _edge_chunk(q):
            j = q * R
            for r in range(R):
                jj = j + r
                bp = (r + R - 1) % R

                @pl.when(jj + R - 1 < nck)
                def _():
                    @pl.when(jj >= 1)
                    def _():
                        pltpu.make_async_copy(bufs[bp], acc.at[dstv.at[0]], ssems[bp]).wait()

                    pltpu.async_copy(g_hbm.at[srcv.at[jj + R - 1]], bufs[bp], gsems[bp])

                pltpu.make_async_copy(g_hbm.at[srcv.at[0]], bufs[r], gsems[r]).wait()
                pltpu.async_copy(bufs[r], acc.at[dstv.at[jj]], ssems[r], add=True)

        for r in range(R):
            pltpu.make_async_copy(bufs[r], acc.at[dstv.at[0]], ssems[r]).wait()

    @pl.when(w < NW - 1)
    def _():
        _ring(CPT, 8)

    @pl.when(w == NW - 1)
    def _():
        _ring(CPT_LAST, 4)

    plsc.subcore_barrier()
    base = s * RPT
    pltpu.sync_copy(acc.at[pl.ds(base, RPT)], out_hbm.at[c].at[pl.ds(base, RPT)])


def _mm1_body(deg_ref, x_ref, w_ref, g_ref, dinv_ref):
    deg = deg_ref[0, :, 0:1] + deg_ref[1, :, 0:1] + 1.0
    dinv = lax.rsqrt(jnp.maximum(deg, 1.0))
    dinv_ref[...] = dinv
    g_ref[...] = jnp.dot(x_ref[...] * dinv, w_ref[...], preferred_element_type=jnp.float32)


def _mm2_body(p_ref, g_ref, dinv_ref, b_ref, w_ref, o_ref):
    dinv = dinv_ref[...]
    t = (p_ref[0] + p_ref[1] - g_ref[...]) * dinv + b_ref[...]
    h = jnp.maximum(t, 0.0)
    o_ref[...] = jnp.dot(h * dinv, w_ref[...], preferred_element_type=jnp.float32)


def _head_body(p_ref, g_ref, dinv_ref, b_ref, wq_ref, bq_ref, o_ref):
    dinv = dinv_ref[...]
    t = (p_ref[0] + p_ref[1] - g_ref[...]) * dinv + b_ref[...]
    h = jnp.maximum(t, 0.0)
    o_ref[...] = jnp.dot(h, wq_ref[...], preferred_element_type=jnp.float32) + bq_ref[...]


def _row_spec(cols):
    return pl.BlockSpec((BR, cols), lambda i: (i, 0))


def _pair_spec(cols):
    return pl.BlockSpec((2, BR, cols), lambda i: (0, i, 0))


def _full_spec(rows, cols):
    return pl.BlockSpec((rows, cols), lambda i: (0, 0))


_mm1_call = pl.pallas_call(
    _mm1_body,
    grid=(N // BR,),
    in_specs=[_pair_spec(L), _row_spec(D_IN), _full_spec(D_IN, H)],
    out_specs=[_row_spec(H), _row_spec(1)],
    out_shape=[
        jax.ShapeDtypeStruct((N, H), jnp.float32),
        jax.ShapeDtypeStruct((N, 1), jnp.float32),
    ],
)

_mm2_call = pl.pallas_call(
    _mm2_body,
    grid=(N // BR,),
    in_specs=[
        _pair_spec(H), _row_spec(H), _row_spec(1),
        _full_spec(1, H), _full_spec(H, H),
    ],
    out_specs=_row_spec(H),
    out_shape=jax.ShapeDtypeStruct((N, H), jnp.float32),
)

_head_call = pl.pallas_call(
    _head_body,
    grid=(N // BR,),
    in_specs=[
        _pair_spec(H), _row_spec(H), _row_spec(1),
        _full_spec(1, H), _full_spec(H, 1), _full_spec(1, 1),
    ],
    out_specs=_row_spec(1),
    out_shape=jax.ShapeDtypeStruct((N, 1), jnp.float32),
)


def kernel(x, edge_index, W1, b1, W2, b2, Wq, bq):
    src2d = edge_index[0].astype(jnp.int32).reshape(EC, CH)
    dst2d = edge_index[1].astype(jnp.int32).reshape(EC, CH)

    zc = jnp.zeros((RPT, L), jnp.float32)
    oc = jnp.ones((CH, L), jnp.float32)
    deg = _hist_kernel(dst2d, zc, oc)
    g1, dinv = _mm1_call(deg, x, W1)
    p = _scatter_kernel(g1, src2d, dst2d)
    g2 = _mm2_call(p, g1, dinv, b1.reshape(1, H), W2)
    p2 = _scatter_kernel(g2, src2d, dst2d)
    q = _head_call(p2, g2, dinv, b2.reshape(1, H), Wq, bq.reshape(1, 1))
    return q[:, 0]


# speedup vs baseline: 1.0050x; 1.0050x over previous
import functools

import jax
import jax.numpy as jnp
from jax import lax
from jax.experimental import pallas as pl
from jax.experimental.pallas import tpu as pltpu
from jax.experimental.pallas import tpu_sc as plsc

N = 10000
E = 320000
D_IN = 128
H = 64

NC = 2
NS = 16
NW = NC * NS
L = 16

CH = 128
EC = E // CH
CPT = 80
CPT_LAST = EC - 31 * CPT
EPT_PAD = CPT * CH

GCH = 512
GPB = GCH // CH

NR = 10240
RPT = NR // NS
IPT = N // NS - 16

BR = N

_mesh = plsc.VectorSubcoreMesh(core_axis_name="c", subcore_axis_name="s")
_sc_params = pltpu.CompilerParams(use_tc_tiling_on_sc=False)


@functools.partial(
    pl.kernel,
    out_type=jax.ShapeDtypeStruct((NC, NR, L), jnp.float32),
    mesh=_mesh,
    compiler_params=_sc_params,
    scratch_types=[
        pltpu.VMEM((CPT, CH), jnp.int32),
        pltpu.VMEM((CH, L), jnp.float32),
        [pltpu.SemaphoreType.DMA for _ in range(4)],
        pltpu.VMEM_SHARED((NR, L), jnp.float32),
    ],
)
def _hist_kernel(dst_hbm, zero_hbm, ones_hbm, out_hbm, dstv, buf, hsems, sdeg):
    c = lax.axis_index("c")
    s = lax.axis_index("s")
    w = s * NC + c
    nch = jnp.where(w == NW - 1, CPT_LAST, CPT)

    @pl.when(w < NW - 1)
    def _():
        pltpu.sync_copy(dst_hbm.at[pl.ds(w * CPT, CPT)], dstv)

    @pl.when(w == NW - 1)
    def _():
        pltpu.sync_copy(dst_hbm.at[pl.ds(w * CPT, CPT_LAST)], dstv.at[pl.ds(0, CPT_LAST)])

    base = s * RPT
    pltpu.sync_copy(zero_hbm, sdeg.at[pl.ds(base, RPT)])
    pltpu.sync_copy(ones_hbm, buf)

    plsc.subcore_barrier()

    @pl.loop(0, nch // 4)
    def _chunkgrp(q):
        j = q * 4
        for r in range(4):
            @pl.when(q > 0)
            def _():
                pltpu.make_async_copy(buf, sdeg.at[dstv.at[0]], hsems[r]).wait()

            pltpu.async_copy(buf, sdeg.at[dstv.at[j + r]], hsems[r], add=True)

    for r in range(4):
        pltpu.make_async_copy(buf, sdeg.at[dstv.at[0]], hsems[r]).wait()

    plsc.subcore_barrier()
    pltpu.sync_copy(sdeg.at[pl.ds(base, RPT)], out_hbm.at[c].at[pl.ds(base, RPT)])


@functools.partial(
    pl.kernel,
    out_type=jax.ShapeDtypeStruct((NC, NR, H), jnp.float32),
    mesh=_mesh,
    compiler_params=_sc_params,
    scratch_types=[
        pltpu.VMEM((CPT, CH), jnp.int32),
        pltpu.VMEM((CPT, CH), jnp.int32),
        [pltpu.VMEM((CH, H), jnp.float32) for _ in range(4)],
        [pltpu.SemaphoreType.DMA for _ in range(4)],
        [pltpu.SemaphoreType.DMA for _ in range(4)],
        pltpu.VMEM_SHARED((NR, H), jnp.float32),
    ],
)
def _scatter_kernel(g_hbm, src_hbm, dst_hbm, out_hbm, srcv, dstv, bufs, gsems, ssems, acc):
    c = lax.axis_index("c")
    s = lax.axis_index("s")
    w = s * NC + c
    nch = jnp.where(w == NW - 1, CPT_LAST, CPT)

    @pl.when(w < NW - 1)
    def _():
        pltpu.sync_copy(src_hbm.at[pl.ds(w * CPT, CPT)], srcv)
        pltpu.sync_copy(dst_hbm.at[pl.ds(w * CPT, CPT)], dstv)

    @pl.when(w == NW - 1)
    def _():
        pltpu.sync_copy(src_hbm.at[pl.ds(w * CPT, CPT_LAST)], srcv.at[pl.ds(0, CPT_LAST)])
        pltpu.sync_copy(dst_hbm.at[pl.ds(w * CPT, CPT_LAST)], dstv.at[pl.ds(0, CPT_LAST)])

    ibase = s * IPT
    pltpu.sync_copy(g_hbm.at[pl.ds(ibase, IPT)], acc.at[pl.ds(ibase, IPT)])

    @pl.when(s == NS - 1)
    def _():
        pltpu.sync_copy(g_hbm.at[pl.ds(NS * IPT, N - NS * IPT)],
                        acc.at[pl.ds(NS * IPT, N - NS * IPT)])

    plsc.subcore_barrier()

    def _ring(nck, R):
        for r in range(R - 1):
            pltpu.async_copy(g_hbm.at[srcv.at[r]], bufs[r], gsems[r])

        @pl.loop(0, nck // R)
        def _edge_chunk(q):
            j = q * R
            for r in range(R):
                jj = j + r
                bp = (r + R - 1) % R

                @pl.when(jj + R - 1 < nck)
                def _():
                    @pl.when(jj >= 1)
                    def _():
                        pltpu.make_async_copy(bufs[bp], acc.at[dstv.at[0]], ssems[bp]).wait()

                    pltpu.async_copy(g_hbm.at[srcv.at[jj + R - 1]], bufs[bp], gsems[bp])

                pltpu.make_async_copy(g_hbm.at[srcv.at[0]], bufs[r], gsems[r]).wait()
                pltpu.async_copy(bufs[r], acc.at[dstv.at[jj]], ssems[r], add=True)

        for r in range(R):
            pltpu.make_async_copy(bufs[r], acc.at[dstv.at[0]], ssems[r]).wait()

    @pl.when(w < NW - 1)
    def _():
        _ring(CPT, 4)

    @pl.when(w == NW - 1)
    def _():
        _ring(CPT_LAST, 4)

    plsc.subcore_barrier()
    base = s * RPT
    pltpu.sync_copy(acc.at[pl.ds(base, RPT)], out_hbm.at[c].at[pl.ds(base, RPT)])


def _mm1_body(deg_ref, x_ref, w_ref, g_ref, dinv_ref):
    deg = deg_ref[0, :, 0:1] + deg_ref[1, :, 0:1] + 1.0
    dinv = lax.rsqrt(jnp.maximum(deg, 1.0))
    dinv_ref[...] = dinv
    g_ref[...] = jnp.dot(x_ref[...] * dinv, w_ref[...], preferred_element_type=jnp.float32)


def _mm2_body(p_ref, g_ref, dinv_ref, b_ref, w_ref, o_ref):
    dinv = dinv_ref[...]
    t = (p_ref[0] + p_ref[1] - g_ref[...]) * dinv + b_ref[...]
    h = jnp.maximum(t, 0.0)
    o_ref[...] = jnp.dot(h * dinv, w_ref[...], preferred_element_type=jnp.float32)


def _head_body(p_ref, g_ref, dinv_ref, b_ref, wq_ref, bq_ref, o_ref):
    dinv = dinv_ref[...]
    t = (p_ref[0] + p_ref[1] - g_ref[...]) * dinv + b_ref[...]
    h = jnp.maximum(t, 0.0)
    o_ref[...] = jnp.dot(h, wq_ref[...], preferred_element_type=jnp.float32) + bq_ref[...]


def _row_spec(cols):
    return pl.BlockSpec((BR, cols), lambda i: (i, 0))


def _pair_spec(cols):
    return pl.BlockSpec((2, BR, cols), lambda i: (0, i, 0))


def _full_spec(rows, cols):
    return pl.BlockSpec((rows, cols), lambda i: (0, 0))


_mm1_call = pl.pallas_call(
    _mm1_body,
    grid=(N // BR,),
    in_specs=[_pair_spec(L), _row_spec(D_IN), _full_spec(D_IN, H)],
    out_specs=[_row_spec(H), _row_spec(1)],
    out_shape=[
        jax.ShapeDtypeStruct((N, H), jnp.float32),
        jax.ShapeDtypeStruct((N, 1), jnp.float32),
    ],
)

_mm2_call = pl.pallas_call(
    _mm2_body,
    grid=(N // BR,),
    in_specs=[
        _pair_spec(H), _row_spec(H), _row_spec(1),
        _full_spec(1, H), _full_spec(H, H),
    ],
    out_specs=_row_spec(H),
    out_shape=jax.ShapeDtypeStruct((N, H), jnp.float32),
)

_head_call = pl.pallas_call(
    _head_body,
    grid=(N // BR,),
    in_specs=[
        _pair_spec(H), _row_spec(H), _row_spec(1),
        _full_spec(1, H), _full_spec(H, 1), _full_spec(1, 1),
    ],
    out_specs=_row_spec(1),
    out_shape=jax.ShapeDtypeStruct((N, 1), jnp.float32),
)


def kernel(x, edge_index, W1, b1, W2, b2, Wq, bq):
    src2d = edge_index[0].astype(jnp.int32).reshape(EC, CH)
    dst2d = edge_index[1].astype(jnp.int32).reshape(EC, CH)

    zc = jnp.zeros((RPT, L), jnp.float32)
    oc = jnp.ones((CH, L), jnp.float32)
    deg = _hist_kernel(dst2d, zc, oc)
    g1, dinv = _mm1_call(deg, x, W1)
    p = _scatter_kernel(g1, src2d, dst2d)
    g2 = _mm2_call(p, g1, dinv, b1.reshape(1, H), W2)
    p2 = _scatter_kernel(g2, src2d, dst2d)
    q = _head_call(p2, g2, dinv, b2.reshape(1, H), Wq, bq.reshape(1, 1))
    return q[:, 0]


# speedup vs baseline: 1.0076x; 1.0026x over previous
import functools

import jax
import jax.numpy as jnp
from jax import lax
from jax.experimental import pallas as pl
from jax.experimental.pallas import tpu as pltpu
from jax.experimental.pallas import tpu_sc as plsc

N = 10000
E = 320000
D_IN = 128
H = 64

NC = 2
NS = 16
NW = NC * NS
L = 16

CH = 128
EC = E // CH
CPT = 80
CPT_LAST = EC - 31 * CPT
EPT_PAD = CPT * CH

GCH = 512
GPB = GCH // CH

NR = 10240
RPT = NR // NS
IPT = N // NS - 16

BR = N

_mesh = plsc.VectorSubcoreMesh(core_axis_name="c", subcore_axis_name="s")
_sc_params = pltpu.CompilerParams(use_tc_tiling_on_sc=False)


@functools.partial(
    pl.kernel,
    out_type=jax.ShapeDtypeStruct((NC, NR, L), jnp.float32),
    mesh=_mesh,
    compiler_params=_sc_params,
    scratch_types=[
        pltpu.VMEM((CPT, CH), jnp.int32),
        pltpu.VMEM((RPT, L), jnp.float32),
        [pltpu.SemaphoreType.DMA for _ in range(4)],
        pltpu.VMEM_SHARED((NR, L), jnp.float32),
    ],
)
def _hist_kernel(dst_hbm, out_hbm, dstv, buf, hsems, sdeg):
    c = lax.axis_index("c")
    s = lax.axis_index("s")
    w = s * NC + c
    nch = jnp.where(w == NW - 1, CPT_LAST, CPT)

    @pl.when(w < NW - 1)
    def _():
        pltpu.sync_copy(dst_hbm.at[pl.ds(w * CPT, CPT)], dstv)

    @pl.when(w == NW - 1)
    def _():
        pltpu.sync_copy(dst_hbm.at[pl.ds(w * CPT, CPT_LAST)], dstv.at[pl.ds(0, CPT_LAST)])

    zero = jnp.zeros((L,), jnp.float32)

    @pl.loop(0, RPT)
    def _zero(i):
        buf[i, pl.ds(0, L)] = zero

    base = s * RPT
    pltpu.sync_copy(buf, sdeg.at[pl.ds(base, RPT)])

    ones = jnp.full((L,), 1.0, jnp.float32)

    @pl.loop(0, CH)
    def _one(i):
        buf[i, pl.ds(0, L)] = ones

    plsc.subcore_barrier()

    @pl.loop(0, nch // 4)
    def _chunkgrp(q):
        j = q * 4
        for r in range(4):
            @pl.when(q > 0)
            def _():
                pltpu.make_async_copy(buf.at[pl.ds(0, CH)], sdeg.at[dstv.at[0]], hsems[r]).wait()

            pltpu.async_copy(buf.at[pl.ds(0, CH)], sdeg.at[dstv.at[j + r]], hsems[r], add=True)

    for r in range(4):
        pltpu.make_async_copy(buf.at[pl.ds(0, CH)], sdeg.at[dstv.at[0]], hsems[r]).wait()

    plsc.subcore_barrier()
    pltpu.sync_copy(sdeg.at[pl.ds(base, RPT)], out_hbm.at[c].at[pl.ds(base, RPT)])


@functools.partial(
    pl.kernel,
    out_type=jax.ShapeDtypeStruct((NC, NR, H), jnp.float32),
    mesh=_mesh,
    compiler_params=_sc_params,
    scratch_types=[
        pltpu.VMEM((CPT, CH), jnp.int32),
        pltpu.VMEM((CPT, CH), jnp.int32),
        [pltpu.VMEM((CH, H), jnp.float32) for _ in range(4)],
        [pltpu.SemaphoreType.DMA for _ in range(4)],
        [pltpu.SemaphoreType.DMA for _ in range(4)],
        pltpu.VMEM_SHARED((NR, H), jnp.float32),
    ],
)
def _scatter_kernel(g_hbm, src_hbm, dst_hbm, out_hbm, srcv, dstv, bufs, gsems, ssems, acc):
    c = lax.axis_index("c")
    s = lax.axis_index("s")
    w = s * NC + c
    nch = jnp.where(w == NW - 1, CPT_LAST, CPT)

    @pl.when(w < NW - 1)
    def _():
        pltpu.sync_copy(src_hbm.at[pl.ds(w * CPT, CPT)], srcv)
        pltpu.sync_copy(dst_hbm.at[pl.ds(w * CPT, CPT)], dstv)

    @pl.when(w == NW - 1)
    def _():
        pltpu.sync_copy(src_hbm.at[pl.ds(w * CPT, CPT_LAST)], srcv.at[pl.ds(0, CPT_LAST)])
        pltpu.sync_copy(dst_hbm.at[pl.ds(w * CPT, CPT_LAST)], dstv.at[pl.ds(0, CPT_LAST)])

    ibase = s * IPT
    pltpu.sync_copy(g_hbm.at[pl.ds(ibase, IPT)], acc.at[pl.ds(ibase, IPT)])

    @pl.when(s == NS - 1)
    def _():
        pltpu.sync_copy(g_hbm.at[pl.ds(NS * IPT, N - NS * IPT)],
                        acc.at[pl.ds(NS * IPT, N - NS * IPT)])

    plsc.subcore_barrier()

    def _ring(nck, R):
        for r in range(R - 1):
            pltpu.async_copy(g_hbm.at[srcv.at[r]], bufs[r], gsems[r])

        @pl.loop(0, nck // R)
        def _edge_chunk(q):
            j = q * R
            for r in range(R):
                jj = j + r
                bp = (r + R - 1) % R

                @pl.when(jj + R - 1 < nck)
                def _():
                    @pl.when(jj >= 1)
                    def _():
                        pltpu.make_async_copy(bufs[bp], acc.at[dstv.at[0]], ssems[bp]).wait()

                    pltpu.async_copy(g_hbm.at[srcv.at[jj + R - 1]], bufs[bp], gsems[bp])

                pltpu.make_async_copy(g_hbm.at[srcv.at[0]], bufs[r], gsems[r]).wait()
                pltpu.async_copy(bufs[r], acc.at[dstv.at[jj]], ssems[r], add=True)

        for r in range(R):
            pltpu.make_async_copy(bufs[r], acc.at[dstv.at[0]], ssems[r]).wait()

    @pl.when(w < NW - 1)
    def _():
        _ring(CPT, 4)

    @pl.when(w == NW - 1)
    def _():
        _ring(CPT_LAST, 4)

    plsc.subcore_barrier()
    base = s * RPT
    pltpu.sync_copy(acc.at[pl.ds(base, RPT)], out_hbm.at[c].at[pl.ds(base, RPT)])


def _mm1_body(deg_ref, x_ref, w_ref, g_ref, dinv_ref):
    deg = deg_ref[0, :, 0:1] + deg_ref[1, :, 0:1] + 1.0
    dinv = lax.rsqrt(jnp.maximum(deg, 1.0))
    dinv_ref[...] = dinv
    g_ref[...] = jnp.dot(x_ref[...] * dinv, w_ref[...], preferred_element_type=jnp.float32)


def _mm2_body(p_ref, g_ref, dinv_ref, b_ref, w_ref, o_ref):
    dinv = dinv_ref[...]
    t = (p_ref[0] + p_ref[1] - g_ref[...]) * dinv + b_ref[...]
    h = jnp.maximum(t, 0.0)
    o_ref[...] = jnp.dot(h * dinv, w_ref[...], preferred_element_type=jnp.float32)


def _head_body(p_ref, g_ref, dinv_ref, b_ref, wq_ref, bq_ref, o_ref):
    dinv = dinv_ref[...]
    t = (p_ref[0] + p_ref[1] - g_ref[...]) * dinv + b_ref[...]
    h = jnp.maximum(t, 0.0)
    o_ref[...] = jnp.dot(h, wq_ref[...], preferred_element_type=jnp.float32) + bq_ref[...]


def _row_spec(cols):
    return pl.BlockSpec((BR, cols), lambda i: (i, 0))


def _pair_spec(cols):
    return pl.BlockSpec((2, BR, cols), lambda i: (0, i, 0))


def _full_spec(rows, cols):
    return pl.BlockSpec((rows, cols), lambda i: (0, 0))


_mm1_call = pl.pallas_call(
    _mm1_body,
    grid=(N // BR,),
    in_specs=[_pair_spec(L), _row_spec(D_IN), _full_spec(D_IN, H)],
    out_specs=[_row_spec(H), _row_spec(1)],
    out_shape=[
        jax.ShapeDtypeStruct((N, H), jnp.float32),
        jax.ShapeDtypeStruct((N, 1), jnp.float32),
    ],
)

_mm2_call = pl.pallas_call(
    _mm2_body,
    grid=(N // BR,),
    in_specs=[
        _pair_spec(H), _row_spec(H), _row_spec(1),
        _full_spec(1, H), _full_spec(H, H),
    ],
    out_specs=_row_spec(H),
    out_shape=jax.ShapeDtypeStruct((N, H), jnp.float32),
)

_head_call = pl.pallas_call(
    _head_body,
    grid=(N // BR,),
    in_specs=[
        _pair_spec(H), _row_spec(H), _row_spec(1),
        _full_spec(1, H), _full_spec(H, 1), _full_spec(1, 1),
    ],
    out_specs=_row_spec(1),
    out_shape=jax.ShapeDtypeStruct((N, 1), jnp.float32),
)


def kernel(x, edge_index, W1, b1, W2, b2, Wq, bq):
    src2d = edge_index[0].astype(jnp.int32).reshape(EC, CH)
    dst2d = edge_index[1].astype(jnp.int32).reshape(EC, CH)

    deg = _hist_kernel(dst2d)
    g1, dinv = _mm1_call(deg, x, W1)
    p = _scatter_kernel(g1, src2d, dst2d)
    g2 = _mm2_call(p, g1, dinv, b1.reshape(1, H), W2)
    p2 = _scatter_kernel(g2, src2d, dst2d)
    q = _head_call(p2, g2, dinv, b2.reshape(1, H), Wq, bq.reshape(1, 1))
    return q[:, 0]


# speedup vs baseline: 1.0087x; 1.0012x over previous
import functools

import jax
import jax.numpy as jnp
from jax import lax
from jax.experimental import pallas as pl
from jax.experimental.pallas import tpu as pltpu
from jax.experimental.pallas import tpu_sc as plsc

N = 10000
E = 320000
D_IN = 128
H = 64

NC = 2
NS = 16
NW = NC * NS
L = 16

CH = 128
EC = E // CH
CPT = 80
CPT_LAST = EC - 31 * CPT
NR = 10240
RPT = NR // NS
IPT = N // NS - 16

BR = N

_mesh = plsc.VectorSubcoreMesh(core_axis_name="c", subcore_axis_name="s")
_sc_params = pltpu.CompilerParams(use_tc_tiling_on_sc=False)


@functools.partial(
    pl.kernel,
    out_type=jax.ShapeDtypeStruct((NC, NR, L), jnp.float32),
    mesh=_mesh,
    compiler_params=_sc_params,
    scratch_types=[
        pltpu.VMEM((CPT, CH), jnp.int32),
        pltpu.VMEM((RPT, L), jnp.float32),
        [pltpu.SemaphoreType.DMA for _ in range(4)],
        pltpu.VMEM_SHARED((NR, L), jnp.float32),
    ],
)
def _hist_kernel(dst_hbm, out_hbm, dstv, buf, hsems, sdeg):
    c = lax.axis_index("c")
    s = lax.axis_index("s")
    w = s * NC + c
    nch = jnp.where(w == NW - 1, CPT_LAST, CPT)

    @pl.when(w < NW - 1)
    def _():
        pltpu.sync_copy(dst_hbm.at[pl.ds(w * CPT, CPT)], dstv)

    @pl.when(w == NW - 1)
    def _():
        pltpu.sync_copy(dst_hbm.at[pl.ds(w * CPT, CPT_LAST)], dstv.at[pl.ds(0, CPT_LAST)])

    zero = jnp.zeros((L,), jnp.float32)

    @pl.loop(0, RPT)
    def _zero(i):
        buf[i, pl.ds(0, L)] = zero

    base = s * RPT
    pltpu.sync_copy(buf, sdeg.at[pl.ds(base, RPT)])

    ones = jnp.full((L,), 1.0, jnp.float32)

    @pl.loop(0, CH)
    def _one(i):
        buf[i, pl.ds(0, L)] = ones

    plsc.subcore_barrier()

    @pl.loop(0, nch // 4)
    def _chunkgrp(q):
        j = q * 4
        for r in range(4):
            @pl.when(q > 0)
            def _():
                pltpu.make_async_copy(buf.at[pl.ds(0, CH)], sdeg.at[dstv.at[0]], hsems[r]).wait()

            pltpu.async_copy(buf.at[pl.ds(0, CH)], sdeg.at[dstv.at[j + r]], hsems[r], add=True)

    for r in range(4):
        pltpu.make_async_copy(buf.at[pl.ds(0, CH)], sdeg.at[dstv.at[0]], hsems[r]).wait()

    plsc.subcore_barrier()
    pltpu.sync_copy(sdeg.at[pl.ds(base, RPT)], out_hbm.at[c].at[pl.ds(base, RPT)])


@functools.partial(
    pl.kernel,
    out_type=jax.ShapeDtypeStruct((NC, NR, H), jnp.float32),
    mesh=_mesh,
    compiler_params=_sc_params,
    scratch_types=[
        pltpu.VMEM((CPT, CH), jnp.int32),
        pltpu.VMEM((CPT, CH), jnp.int32),
        [pltpu.VMEM((CH, H), jnp.float32) for _ in range(4)],
        [pltpu.SemaphoreType.DMA for _ in range(4)],
        [pltpu.SemaphoreType.DMA for _ in range(4)],
        pltpu.VMEM_SHARED((NR, H), jnp.float32),
    ],
)
def _scatter_kernel(g_hbm, src_hbm, dst_hbm, out_hbm, srcv, dstv, bufs, gsems, ssems, acc):
    c = lax.axis_index("c")
    s = lax.axis_index("s")
    w = s * NC + c

    @pl.when(w < NW - 1)
    def _():
        pltpu.sync_copy(src_hbm.at[pl.ds(w * CPT, CPT)], srcv)
        pltpu.sync_copy(dst_hbm.at[pl.ds(w * CPT, CPT)], dstv)

    @pl.when(w == NW - 1)
    def _():
        pltpu.sync_copy(src_hbm.at[pl.ds(w * CPT, CPT_LAST)], srcv.at[pl.ds(0, CPT_LAST)])
        pltpu.sync_copy(dst_hbm.at[pl.ds(w * CPT, CPT_LAST)], dstv.at[pl.ds(0, CPT_LAST)])

    ibase = s * IPT
    pltpu.sync_copy(g_hbm.at[pl.ds(ibase, IPT)], acc.at[pl.ds(ibase, IPT)])

    @pl.when(s == NS - 1)
    def _():
        pltpu.sync_copy(g_hbm.at[pl.ds(NS * IPT, N - NS * IPT)],
                        acc.at[pl.ds(NS * IPT, N - NS * IPT)])

    plsc.subcore_barrier()

    def _ring(nck, R):
        for r in range(R - 1):
            pltpu.async_copy(g_hbm.at[srcv.at[r]], bufs[r], gsems[r])

        @pl.loop(0, nck // R)
        def _edge_chunk(q):
            j = q * R
            for r in range(R):
                jj = j + r
                bp = (r + R - 1) % R

                @pl.when(jj + R - 1 < nck)
                def _():
                    @pl.when(jj >= 1)
                    def _():
                        pltpu.make_async_copy(bufs[bp], acc.at[dstv.at[0]], ssems[bp]).wait()

                    pltpu.async_copy(g_hbm.at[srcv.at[jj + R - 1]], bufs[bp], gsems[bp])

                pltpu.make_async_copy(g_hbm.at[srcv.at[0]], bufs[r], gsems[r]).wait()
                pltpu.async_copy(bufs[r], acc.at[dstv.at[jj]], ssems[r], add=True)

        for r in range(R):
            pltpu.make_async_copy(bufs[r], acc.at[dstv.at[0]], ssems[r]).wait()

    @pl.when(w < NW - 1)
    def _():
        _ring(CPT, 4)

    @pl.when(w == NW - 1)
    def _():
        _ring(CPT_LAST, 4)

    plsc.subcore_barrier()
    base = s * RPT
    pltpu.sync_copy(acc.at[pl.ds(base, RPT)], out_hbm.at[c].at[pl.ds(base, RPT)])


def _mm1_body(deg_ref, x_ref, w_ref, g_ref, dinv_ref):
    deg = deg_ref[0, :, 0:1] + deg_ref[1, :, 0:1] + 1.0
    dinv = lax.rsqrt(jnp.maximum(deg, 1.0))
    dinv_ref[...] = dinv
    g_ref[...] = jnp.dot(x_ref[...] * dinv, w_ref[...], preferred_element_type=jnp.float32)


def _mm2_body(p_ref, g_ref, dinv_ref, b_ref, w_ref, o_ref):
    dinv = dinv_ref[...]
    t = (p_ref[0] + p_ref[1] - g_ref[...]) * dinv + b_ref[...]
    h = jnp.maximum(t, 0.0)
    o_ref[...] = jnp.dot(h * dinv, w_ref[...], preferred_element_type=jnp.float32)


def _head_body(p_ref, g_ref, dinv_ref, b_ref, wq_ref, bq_ref, o_ref):
    dinv = dinv_ref[...]
    t = (p_ref[0] + p_ref[1] - g_ref[...]) * dinv + b_ref[...]
    h = jnp.maximum(t, 0.0)
    o_ref[...] = jnp.dot(h, wq_ref[...], preferred_element_type=jnp.float32) + bq_ref[...]


def _row_spec(cols):
    return pl.BlockSpec((BR, cols), lambda i: (i, 0))


def _pair_spec(cols):
    return pl.BlockSpec((2, BR, cols), lambda i: (0, i, 0))


def _full_spec(rows, cols):
    return pl.BlockSpec((rows, cols), lambda i: (0, 0))


_mm1_call = pl.pallas_call(
    _mm1_body,
    grid=(N // BR,),
    in_specs=[_pair_spec(L), _row_spec(D_IN), _full_spec(D_IN, H)],
    out_specs=[_row_spec(H), _row_spec(1)],
    out_shape=[
        jax.ShapeDtypeStruct((N, H), jnp.float32),
        jax.ShapeDtypeStruct((N, 1), jnp.float32),
    ],
)

_mm2_call = pl.pallas_call(
    _mm2_body,
    grid=(N // BR,),
    in_specs=[
        _pair_spec(H), _row_spec(H), _row_spec(1),
        _full_spec(1, H), _full_spec(H, H),
    ],
    out_specs=_row_spec(H),
    out_shape=jax.ShapeDtypeStruct((N, H), jnp.float32),
)

_head_call = pl.pallas_call(
    _head_body,
    grid=(N // BR,),
    in_specs=[
        _pair_spec(H), _row_spec(H), _row_spec(1),
        _full_spec(1, H), _full_spec(H, 1), _full_spec(1, 1),
    ],
    out_specs=_row_spec(1),
    out_shape=jax.ShapeDtypeStruct((N, 1), jnp.float32),
)


def kernel(x, edge_index, W1, b1, W2, b2, Wq, bq):
    src2d = edge_index[0].astype(jnp.int32).reshape(EC, CH)
    dst2d = edge_index[1].astype(jnp.int32).reshape(EC, CH)

    deg = _hist_kernel(dst2d)
    g1, dinv = _mm1_call(deg, x, W1)
    p = _scatter_kernel(g1, src2d, dst2d)
    g2 = _mm2_call(p, g1, dinv, b1.reshape(1, H), W2)
    p2 = _scatter_kernel(g2, src2d, dst2d)
    q = _head_call(p2, g2, dinv, b2.reshape(1, H), Wq, bq.reshape(1, 1))
    return q[:, 0]
